# baseline (speedup 1.0000x reference)
import functools

import numpy as np
import jax
import jax.numpy as jnp
from jax import lax
from jax.experimental import pallas as pl
from jax.experimental.pallas import tpu as pltpu
from jax.experimental.pallas import tpu_sc as plsc

_M = 512
_C = 64
_ROWS = _M + 2
_ROWS_PAD = 520
_L = 16


def _solve_constant():
    M = _M
    delta = 1.0 / (M - 1)
    dis = (1.0 / delta) ** 2
    A = np.zeros((M + 2, M + 2), dtype=np.float64)
    A[0, 0] = dis
    A[0, 1] = -2.0 * dis
    A[0, 2] = dis
    di = np.arange(1, M + 1)
    A[di, di - 1] = 1.0 / 6.0
    A[di, di] = 2.0 / 3.0
    A[di, di + 1] = 1.0 / 6.0
    A[M + 1, M - 1] = dis
    A[M + 1, M] = -2.0 * dis
    A[M + 1, M + 1] = dis
    K = np.linalg.inv(A)[:, 1:M + 1]
    Kp = np.zeros((_ROWS_PAD, M), dtype=np.float32)
    Kp[:_ROWS, :] = K.astype(np.float32)
    return Kp


_K_CONST = _solve_constant()


def _chtab():
    lane = np.arange(_L, dtype=np.int32)
    c = np.arange(_C, dtype=np.int32)
    return jnp.asarray(((lane[None, :] + c[:, None]) & (_C - 1)).reshape(-1))


def _coefs_body(k_ref, data_ref, out_ref):
    out_ref[...] = lax.dot_general(
        k_ref[...], data_ref[...],
        (((1,), (1,)), ((), ())),
        preferred_element_type=jnp.float32,
    )


def _compute_table(data):
    return pl.pallas_call(
        _coefs_body,
        out_shape=jax.ShapeDtypeStruct((_ROWS_PAD, _C), jnp.float32),
    )(jnp.asarray(_K_CONST), data)


_NC = 2
_NS = 16
_NW = _NC * _NS
_CH = 256
_DO_COMPUTE = True
_PROBE_NO_GATHER = True
_RS = 64


def _sc_interpolate(u_flat, table_flat):
    n = u_flat.shape[0]
    per_w = n // _NW
    n_chunks = per_w // _CH
    tab_words = _ROWS_PAD * _RS
    mesh = plsc.VectorSubcoreMesh(core_axis_name="c", subcore_axis_name="s")

    tb_words = tab_words - 2 * _RS
    scratch = [
        pltpu.VMEM((tab_words,), jnp.float32),
        pltpu.VMEM((tb_words,), jnp.float32),
        pltpu.VMEM((_C * _L,), jnp.int32),
        pltpu.VMEM((_CH,), jnp.float32),
        pltpu.VMEM((_CH,), jnp.float32),
        pltpu.VMEM((_CH * _C,), jnp.float32),
        pltpu.VMEM((_CH * _C,), jnp.float32),
        pltpu.SemaphoreType.DMA,
        pltpu.SemaphoreType.DMA,
        pltpu.SemaphoreType.DMA,
        pltpu.SemaphoreType.DMA,
    ]

    @functools.partial(
        pl.kernel, mesh=mesh,
        out_type=jax.ShapeDtypeStruct((n * _C,), jnp.float32),
        compiler_params=pltpu.CompilerParams(
            needs_layout_passes=False,
            disable_bounds_checks=True,
        ),
        scratch_types=scratch,
    )
    def body(u_hbm, tab_hbm, ch_hbm, out_hbm, tab_v, tb_v, ch_v,
             u0, u1, o0, o1, su0, su1, so0, so1):
        wid = lax.axis_index("s") * _NC + lax.axis_index("c")
        pltpu.sync_copy(tab_hbm, tab_v)
        pltpu.sync_copy(tab_hbm.at[pl.ds(2 * _RS, tb_words)], tb_v)
        pltpu.sync_copy(ch_hbm, ch_v)
        base_pt = wid * per_w
        lane = lax.iota(jnp.int32, _L)
        lane64 = lane * _C
        u_bufs, o_bufs = (u0, u1), (o0, o1)
        su, so = (su0, su1), (so0, so1)

        def u_src(ci):
            return u_hbm.at[pl.ds(base_pt + ci * _CH, _CH)]

        def o_dst(ci):
            return out_hbm.at[pl.ds((base_pt + ci * _CH) * _C, _CH * _C)]

        def make_group_body(u_v, o_v):
            def group_body(g):
                uu = u_v[pl.ds(g * _L, _L)]
                un = uu * jnp.float32(_M - 1)
                ii = un.astype(jnp.int32)
                ii = jnp.minimum(jnp.maximum(ii, 0), _M - 2)
                t = un - ii.astype(jnp.float32)
                t2 = t * t
                t3 = t2 * t
                sixth = jnp.float32(1.0 / 6.0)
                w0 = (((3.0 - t) * t - 3.0) * t + 1.0) * sixth
                w1 = ((3.0 * t - 6.0) * t2 + 4.0) * sixth
                w2 = (((3.0 - 3.0 * t) * t + 3.0) * t + 1.0) * sixth
                w3 = t3 * sixth
                idx0 = ii * _RS
                sbase = lane64 + g * (_L * _C)
                for c in range(_C):
                    ch = ch_v[pl.ds(c * _L, _L)]
                    g0 = idx0 + ch
                    g1 = g0 + _RS
                    acc = w0 * plsc.load_gather(tab_v, [g0])
                    acc = acc + w1 * plsc.load_gather(tab_v, [g1])
                    acc = acc + w2 * plsc.load_gather(tb_v, [g0])
                    acc = acc + w3 * plsc.load_gather(tb_v, [g1])
                    plsc.store_scatter(o_v, [sbase + ch], acc)
            return group_body

        pltpu.async_copy(u_src(0), u0, su0)
        pltpu.async_copy(u_src(1), u1, su1)

        def chunk_pair(ci0, _):
            for b in range(2):
                ci = ci0 + b
                pltpu.make_async_copy(u_src(ci), u_bufs[b], su[b]).wait()

                @pl.when(ci >= 2)
                def _():
                    pltpu.make_async_copy(o_bufs[b], o_dst(ci - 2), so[b]).wait()

                plsc.parallel_loop(0, _CH // _L, unroll=1)(
                    make_group_body(u_bufs[b], o_bufs[b]))
                pltpu.async_copy(o_bufs[b], o_dst(ci), so[b])

                @pl.when(ci + 2 < n_chunks)
                def _():
                    pltpu.async_copy(u_src(ci + 2), u_bufs[b], su[b])
            return 0

        lax.fori_loop(0, n_chunks // 2, lambda i, c: chunk_pair(i * 2, c), 0)
        for b in range(2):
            pltpu.make_async_copy(o_bufs[b], o_dst(n_chunks - 2 + b), so[b]).wait()

    return body(u_flat, table_flat, _chtab())


def kernel(u, data):
    u_flat = u.reshape(-1)
    table = _compute_table(data)
    out_flat = _sc_interpolate(u_flat, table.reshape(-1))
    return out_flat.reshape(u_flat.shape[0], _C)


# speedup vs baseline: 1.1593x; 1.1593x over previous
import functools

import numpy as np
import jax
import jax.numpy as jnp
from jax import lax
from jax.experimental import pallas as pl
from jax.experimental.pallas import tpu as pltpu
from jax.experimental.pallas import tpu_sc as plsc

_M = 512
_C = 64
_ROWS = _M + 2
_ROWS_PAD = 520
_L = 16


def _solve_constant():
    M = _M
    delta = 1.0 / (M - 1)
    dis = (1.0 / delta) ** 2
    A = np.zeros((M + 2, M + 2), dtype=np.float64)
    A[0, 0] = dis
    A[0, 1] = -2.0 * dis
    A[0, 2] = dis
    di = np.arange(1, M + 1)
    A[di, di - 1] = 1.0 / 6.0
    A[di, di] = 2.0 / 3.0
    A[di, di + 1] = 1.0 / 6.0
    A[M + 1, M - 1] = dis
    A[M + 1, M] = -2.0 * dis
    A[M + 1, M + 1] = dis
    K = np.linalg.inv(A)[:, 1:M + 1]
    Kp = np.zeros((_ROWS_PAD, M), dtype=np.float32)
    Kp[:_ROWS, :] = K.astype(np.float32)
    return Kp


_K_CONST = _solve_constant()


def _coefs_body(k_ref, data_ref, out_ref):
    out_ref[...] = lax.dot_general(
        k_ref[...], data_ref[...],
        (((1,), (1,)), ((), ())),
        preferred_element_type=jnp.float32,
    )


def _compute_table(data):
    return pl.pallas_call(
        _coefs_body,
        out_shape=jax.ShapeDtypeStruct((_ROWS_PAD, _C), jnp.float32),
    )(jnp.asarray(_K_CONST), data)


_NC = 2
_NS = 16
_NW = _NC * _NS
_CH = 1024
_RS = 64


def _sc_interpolate(u_flat, table_flat):
    n = u_flat.shape[0]
    per_w = n // _NW
    n_chunks = per_w // _CH
    tab_words = _ROWS_PAD * _RS
    mesh = plsc.VectorSubcoreMesh(core_axis_name="c", subcore_axis_name="s")

    scratch = [
        pltpu.VMEM((tab_words,), jnp.float32),
        pltpu.VMEM((_CH,), jnp.float32),
        pltpu.VMEM((_CH * _C,), jnp.float32),
    ]

    @functools.partial(
        pl.kernel, mesh=mesh,
        out_type=jax.ShapeDtypeStruct((n * _C,), jnp.float32),
        compiler_params=pltpu.CompilerParams(
            needs_layout_passes=False,
            disable_bounds_checks=True,
        ),
        scratch_types=scratch,
    )
    def body(u_hbm, tab_hbm, out_hbm, tab_v, u_v, o_v):
        wid = lax.axis_index("s") * _NC + lax.axis_index("c")
        pltpu.sync_copy(tab_hbm, tab_v)
        base_pt = wid * per_w
        lane = lax.iota(jnp.int32, _L)
        lane64 = lane * _C

        def group_body(g):
            uu = u_v[pl.ds(g * _L, _L)]
            un = uu * jnp.float32(_M - 1)
            ii = un.astype(jnp.int32)
            ii = jnp.minimum(jnp.maximum(ii, 0), _M - 2)
            t = un - ii.astype(jnp.float32)
            t2 = t * t
            t3 = t2 * t
            sixth = jnp.float32(1.0 / 6.0)
            w0 = (((3.0 - t) * t - 3.0) * t + 1.0) * sixth
            w1 = ((3.0 * t - 6.0) * t2 + 4.0) * sixth
            w2 = (((3.0 - 3.0 * t) * t + 3.0) * t + 1.0) * sixth
            w3 = t3 * sixth
            idx0 = ii * _RS
            sbase = lane64 + g * (_L * _C)
            for c in range(_C):
                ch = jnp.bitwise_and(lane + c, _C - 1)
                g0 = idx0 + ch
                acc = w0 * plsc.load_gather(tab_v, [g0])
                acc = acc + w1 * plsc.load_gather(tab_v, [g0 + _RS])
                acc = acc + w2 * plsc.load_gather(tab_v, [g0 + 2 * _RS])
                acc = acc + w3 * plsc.load_gather(tab_v, [g0 + 3 * _RS])
                plsc.store_scatter(o_v, [sbase + ch], acc)

        def chunk_body(ci, carry):
            cbase = base_pt + ci * _CH
            pltpu.sync_copy(u_hbm.at[pl.ds(cbase, _CH)], u_v)
            plsc.parallel_loop(0, _CH // _L, unroll=1)(group_body)
            pltpu.sync_copy(o_v, out_hbm.at[pl.ds(cbase * _C, _CH * _C)])
            return carry

        lax.fori_loop(0, n_chunks, chunk_body, 0)

    return body(u_flat, table_flat)


def kernel(u, data):
    u_flat = u.reshape(-1)
    table = _compute_table(data)
    out_flat = _sc_interpolate(u_flat, table.reshape(-1))
    return out_flat.reshape(u_flat.shape[0], _C)


# speedup vs baseline: 2.1547x; 1.8587x over previous
import functools

import numpy as np
import jax
import jax.numpy as jnp
from jax import lax
from jax.experimental import pallas as pl
from jax.experimental.pallas import tpu as pltpu
from jax.experimental.pallas import tpu_sc as plsc

_M = 512
_C = 64
_CP = _C // 2
_ROWS = _M + 2
_ROWS_PAD = 520
_L = 16


def _solve_constant():
    M = _M
    delta = 1.0 / (M - 1)
    dis = (1.0 / delta) ** 2
    A = np.zeros((M + 2, M + 2), dtype=np.float64)
    A[0, 0] = dis
    A[0, 1] = -2.0 * dis
    A[0, 2] = dis
    di = np.arange(1, M + 1)
    A[di, di - 1] = 1.0 / 6.0
    A[di, di] = 2.0 / 3.0
    A[di, di + 1] = 1.0 / 6.0
    A[M + 1, M - 1] = dis
    A[M + 1, M] = -2.0 * dis
    A[M + 1, M + 1] = dis
    K = np.linalg.inv(A)[:, 1:M + 1]
    Kp = np.zeros((_ROWS_PAD, M), dtype=np.float32)
    Kp[:_ROWS, :] = K.astype(np.float32)
    return Kp


_K_CONST = _solve_constant()


def _coefs_body(k_ref, data_ref, out_ref):
    out_ref[...] = lax.dot_general(
        k_ref[...], data_ref[...],
        (((1,), (1,)), ((), ())),
        preferred_element_type=jnp.float32,
    )


def _compute_table(data):
    return pl.pallas_call(
        _coefs_body,
        out_shape=jax.ShapeDtypeStruct((_ROWS_PAD, _C), jnp.float32),
    )(jnp.asarray(_K_CONST), data)


_NC = 2
_NS = 16
_NW = _NC * _NS
_CH = 512
_RS = _CP


def _sc_interpolate(u_flat, ptab_flat):
    n = u_flat.shape[0]
    per_w = n // _NW
    n_chunks = per_w // _CH
    ta_words = _ROWS_PAD * _RS
    tb_words = (_ROWS_PAD - 2) * _RS
    mesh = plsc.VectorSubcoreMesh(core_axis_name="c", subcore_axis_name="s")

    scratch = [
        pltpu.VMEM((ta_words,), jnp.float32),
        pltpu.VMEM((tb_words,), jnp.float32),
        pltpu.VMEM((_CH,), jnp.float32),
        pltpu.VMEM((_CH,), jnp.float32),
        pltpu.VMEM((_CH * _C,), jnp.float32),
        pltpu.VMEM((_CH * _C,), jnp.float32),
        pltpu.SemaphoreType.DMA,
        pltpu.SemaphoreType.DMA,
        pltpu.SemaphoreType.DMA,
        pltpu.SemaphoreType.DMA,
    ]

    @functools.partial(
        pl.kernel, mesh=mesh,
        out_type=jax.ShapeDtypeStruct((n * _C,), jnp.float32),
        compiler_params=pltpu.CompilerParams(
            needs_layout_passes=False,
            disable_bounds_checks=True,
        ),
        scratch_types=scratch,
    )
    def body(u_hbm, tab_hbm, out_hbm, ta_v, tb_v,
             u0, u1, o0, o1, su0, su1, so0, so1):
        wid = lax.axis_index("s") * _NC + lax.axis_index("c")
        pltpu.sync_copy(tab_hbm, ta_v)
        pltpu.sync_copy(tab_hbm.at[pl.ds(2 * _RS, tb_words)], tb_v)
        base_pt = wid * per_w
        lane = lax.iota(jnp.int32, _L)
        lane64 = lane * _C
        u_bufs, o_bufs = (u0, u1), (o0, o1)
        su, so = (su0, su1), (so0, so1)
        ilv = plsc.PackFormat.INTERLEAVED

        def u_src(ci):
            return u_hbm.at[pl.ds(base_pt + ci * _CH, _CH)]

        def o_dst(ci):
            return out_hbm.at[pl.ds((base_pt + ci * _CH) * _C, _CH * _C)]

        def make_group_body(u_v, o_v):
            def group_body(g):
                uu = u_v[pl.ds(g * _L, _L)]
                un = uu * jnp.float32(_M - 1)
                ii = un.astype(jnp.int32)
                ii = jnp.minimum(jnp.maximum(ii, 0), _M - 2)
                t = un - ii.astype(jnp.float32)
                t2 = t * t
                t3 = t2 * t
                sixth = jnp.float32(1.0 / 6.0)
                w0 = (((3.0 - t) * t - 3.0) * t + 1.0) * sixth
                w1 = ((3.0 * t - 6.0) * t2 + 4.0) * sixth
                w2 = (((3.0 - 3.0 * t) * t + 3.0) * t + 1.0) * sixth
                w3 = t3 * sixth
                wp0 = plsc.pack(w0, w0, format=ilv)
                wp1 = plsc.pack(w1, w1, format=ilv)
                wp2 = plsc.pack(w2, w2, format=ilv)
                wp3 = plsc.pack(w3, w3, format=ilv)
                idx0 = ii * _RS
                sbase = lane64 + g * (_L * _C)
                for p in range(_CP):
                    cp = jnp.bitwise_and(lane + p, _CP - 1)
                    g0 = idx0 + cp
                    g1 = g0 + _RS
                    a0 = plsc.bitcast(plsc.load_gather(ta_v, [g0]), jnp.bfloat16)
                    a1 = plsc.bitcast(plsc.load_gather(ta_v, [g1]), jnp.bfloat16)
                    b0 = plsc.bitcast(plsc.load_gather(tb_v, [g0]), jnp.bfloat16)
                    b1 = plsc.bitcast(plsc.load_gather(tb_v, [g1]), jnp.bfloat16)
                    acc = wp0 * a0 + wp1 * a1 + wp2 * b0 + wp3 * b1
                    lo, hi = plsc.unpack(acc, format=ilv)
                    se = sbase + (cp + cp)
                    plsc.store_scatter(o_v, [se], lo)
                    plsc.store_scatter(o_v, [se + 1], hi)
            return group_body

        pltpu.async_copy(u_src(0), u0, su0)
        pltpu.async_copy(u_src(1), u1, su1)

        def chunk_pair(ci0, _):
            for b in range(2):
                ci = ci0 + b
                pltpu.make_async_copy(u_src(ci), u_bufs[b], su[b]).wait()

                @pl.when(ci >= 2)
                def _():
                    pltpu.make_async_copy(o_bufs[b], o_dst(ci - 2), so[b]).wait()

                plsc.parallel_loop(0, _CH // _L, unroll=1)(
                    make_group_body(u_bufs[b], o_bufs[b]))
                pltpu.async_copy(o_bufs[b], o_dst(ci), so[b])

                @pl.when(ci + 2 < n_chunks)
                def _():
                    pltpu.async_copy(u_src(ci + 2), u_bufs[b], su[b])
            return 0

        lax.fori_loop(0, n_chunks // 2, lambda i, c: chunk_pair(i * 2, c), 0)
        for b in range(2):
            pltpu.make_async_copy(o_bufs[b], o_dst(n_chunks - 2 + b), so[b]).wait()

    return body(u_flat, ptab_flat)


def kernel(u, data):
    u_flat = u.reshape(-1)
    table = _compute_table(data)
    tb16 = table.astype(jnp.bfloat16).reshape(_ROWS_PAD, _CP, 2)
    ptab = lax.bitcast_convert_type(tb16, jnp.float32)
    out_flat = _sc_interpolate(u_flat, ptab.reshape(-1))
    return out_flat.reshape(u_flat.shape[0], _C)


# speedup vs baseline: 2.3279x; 1.0804x over previous
import functools

import numpy as np
import jax
import jax.numpy as jnp
from jax import lax
from jax.experimental import pallas as pl
from jax.experimental.pallas import tpu as pltpu
from jax.experimental.pallas import tpu_sc as plsc

_M = 512
_C = 64
_CP = _C // 2
_ROWS = _M + 2
_ROWS_PAD = 520
_L = 16


def _solve_constant():
    M = _M
    delta = 1.0 / (M - 1)
    dis = (1.0 / delta) ** 2
    A = np.zeros((M + 2, M + 2), dtype=np.float64)
    A[0, 0] = dis
    A[0, 1] = -2.0 * dis
    A[0, 2] = dis
    di = np.arange(1, M + 1)
    A[di, di - 1] = 1.0 / 6.0
    A[di, di] = 2.0 / 3.0
    A[di, di + 1] = 1.0 / 6.0
    A[M + 1, M - 1] = dis
    A[M + 1, M] = -2.0 * dis
    A[M + 1, M + 1] = dis
    K = np.linalg.inv(A)[:, 1:M + 1]
    Kp = np.zeros((_ROWS_PAD, M), dtype=np.float32)
    Kp[:_ROWS, :] = K.astype(np.float32)
    return Kp


_K_CONST = _solve_constant()


def _coefs_body(k_ref, data_ref, out_ref):
    out_ref[...] = lax.dot_general(
        k_ref[...], data_ref[...],
        (((1,), (1,)), ((), ())),
        preferred_element_type=jnp.float32,
    )


def _compute_table(data):
    return pl.pallas_call(
        _coefs_body,
        out_shape=jax.ShapeDtypeStruct((_ROWS_PAD, _C), jnp.float32),
    )(jnp.asarray(_K_CONST), data)


_NC = 2
_NS = 16
_NW = _NC * _NS
_CH = 512
_RS = _CP


def _sc_interpolate(u_flat, ptab_flat):
    n = u_flat.shape[0]
    per_w = n // _NW
    n_chunks = per_w // _CH
    ta_words = _ROWS_PAD * _RS
    tb_words = (_ROWS_PAD - 2) * _RS
    mesh = plsc.VectorSubcoreMesh(core_axis_name="c", subcore_axis_name="s")

    scratch = [
        pltpu.VMEM((ta_words,), jnp.float32),
        pltpu.VMEM((tb_words,), jnp.float32),
        pltpu.VMEM((_CH,), jnp.float32),
        pltpu.VMEM((_CH,), jnp.float32),
        pltpu.VMEM((_CH * _C,), jnp.float32),
        pltpu.VMEM((_CH * _C,), jnp.float32),
        pltpu.SemaphoreType.DMA,
        pltpu.SemaphoreType.DMA,
        pltpu.SemaphoreType.DMA,
        pltpu.SemaphoreType.DMA,
    ]

    @functools.partial(
        pl.kernel, mesh=mesh,
        out_type=jax.ShapeDtypeStruct((n * _C,), jnp.float32),
        compiler_params=pltpu.CompilerParams(
            needs_layout_passes=False,
            disable_bounds_checks=True,
        ),
        scratch_types=scratch,
    )
    def body(u_hbm, tab_hbm, out_hbm, ta_v, tb_v,
             u0, u1, o0, o1, su0, su1, so0, so1):
        wid = lax.axis_index("s") * _NC + lax.axis_index("c")
        pltpu.sync_copy(tab_hbm, ta_v)
        pltpu.sync_copy(tab_hbm.at[pl.ds(2 * _RS, tb_words)], tb_v)
        base_pt = wid * per_w
        lane = lax.iota(jnp.int32, _L)
        lane64 = lane * _C
        u_bufs, o_bufs = (u0, u1), (o0, o1)
        su, so = (su0, su1), (so0, so1)
        ilv = plsc.PackFormat.INTERLEAVED

        def u_src(ci):
            return u_hbm.at[pl.ds(base_pt + ci * _CH, _CH)]

        def o_dst(ci):
            return out_hbm.at[pl.ds((base_pt + ci * _CH) * _C, _CH * _C)]

        def make_group_body(u_v, o_v):
            def group_body(g):
                uu = u_v[pl.ds(g * _L, _L)]
                un = uu * jnp.float32(_M - 1)
                ii = un.astype(jnp.int32)
                ii = jnp.minimum(jnp.maximum(ii, 0), _M - 2)
                t = un - ii.astype(jnp.float32)
                t2 = t * t
                t3 = t2 * t
                sixth = jnp.float32(1.0 / 6.0)
                w0 = (((3.0 - t) * t - 3.0) * t + 1.0) * sixth
                w1 = ((3.0 * t - 6.0) * t2 + 4.0) * sixth
                w2 = (((3.0 - 3.0 * t) * t + 3.0) * t + 1.0) * sixth
                w3 = t3 * sixth
                wp0 = plsc.pack(w0, w0, format=ilv)
                wp1 = plsc.pack(w1, w1, format=ilv)
                wp2 = plsc.pack(w2, w2, format=ilv)
                wp3 = plsc.pack(w3, w3, format=ilv)
                idx0 = ii * _RS
                sbase = lane64 + g * (_L * _C)
                for p in range(_CP):
                    cp = jnp.bitwise_and(lane + p, _CP - 1)
                    g0 = idx0 + cp
                    g1 = g0 + _RS
                    a0 = plsc.bitcast(plsc.load_gather(ta_v, [g0]), jnp.bfloat16)
                    a1 = plsc.bitcast(plsc.load_gather(ta_v, [g1]), jnp.bfloat16)
                    b0 = plsc.bitcast(plsc.load_gather(tb_v, [g0]), jnp.bfloat16)
                    b1 = plsc.bitcast(plsc.load_gather(tb_v, [g1]), jnp.bfloat16)
                    acc = wp0 * a0 + wp1 * a1 + wp2 * b0 + wp3 * b1
                    lo, hi = plsc.unpack(acc, format=ilv)
                    se = sbase + (cp + cp)
                    plsc.store_scatter(o_v, [se], lo)
                    plsc.store_scatter(o_v, [se + 1], hi)
            return group_body

        pltpu.async_copy(u_src(0), u0, su0)
        pltpu.async_copy(u_src(1), u1, su1)

        def chunk_pair(ci0, _):
            for b in range(2):
                ci = ci0 + b
                pltpu.make_async_copy(u_src(ci), u_bufs[b], su[b]).wait()

                @pl.when(ci >= 2)
                def _():
                    pltpu.make_async_copy(o_bufs[b], o_dst(ci - 2), so[b]).wait()

                plsc.parallel_loop(0, _CH // _L, unroll=2)(
                    make_group_body(u_bufs[b], o_bufs[b]))
                pltpu.async_copy(o_bufs[b], o_dst(ci), so[b])

                @pl.when(ci + 2 < n_chunks)
                def _():
                    pltpu.async_copy(u_src(ci + 2), u_bufs[b], su[b])
            return 0

        lax.fori_loop(0, n_chunks // 2, lambda i, c: chunk_pair(i * 2, c), 0)
        for b in range(2):
            pltpu.make_async_copy(o_bufs[b], o_dst(n_chunks - 2 + b), so[b]).wait()

    return body(u_flat, ptab_flat)


def kernel(u, data):
    u_flat = u.reshape(-1)
    table = _compute_table(data)
    tb16 = table.astype(jnp.bfloat16).reshape(_ROWS_PAD, _CP, 2)
    ptab = lax.bitcast_convert_type(tb16, jnp.float32)
    out_flat = _sc_interpolate(u_flat, ptab.reshape(-1))
    return out_flat.reshape(u_flat.shape[0], _C)
